# baseline (speedup 1.0000x reference)
import jax
import jax.numpy as jnp
from jax import lax
from jax.experimental import pallas as pl
from jax.experimental.pallas import tpu as pltpu
from jax.experimental.pallas import tpu_sc as plsc

VOCAB = 1000000
EMB = 32
BATCH = 16384

NC = 2
NS = 16
L = 16
NW = NC * NS
BPW = BATCH // NW
CHUNK = 128
NCH = BPW // CHUNK
G = BPW // L

RBLK = 8192


def _repack(v):
    bits = jax.lax.bitcast_convert_type(v, jnp.int32)
    rnd = (bits + 0x7FFF + ((bits >> 16) & 1)) >> 16
    lo = rnd[:, : EMB // 2] & 0xFFFF
    hi = rnd[:, EMB // 2 :] << 16
    return hi | lo


def _sc_body(c_hbm, t_hbm, co_hbm, wt_hbm, v_hbm, u_hbm, vb_hbm, ub_hbm,
             out_hbm, idx_c, idx_t, rows_v, rows_u, vbv, ubv, cov, wtv,
             accv, sem):
    wid = lax.axis_index("s") * NC + lax.axis_index("c")

    pltpu.sync_copy(c_hbm.at[wid], idx_c)
    pltpu.sync_copy(t_hbm.at[wid], idx_t)

    copies = []
    for j in range(NCH):
        sl = pl.ds(j * CHUNK, CHUNK)
        copies.append(pltpu.async_copy(v_hbm.at[idx_c.at[j]], rows_v.at[sl], sem))
        copies.append(pltpu.async_copy(u_hbm.at[idx_t.at[j]], rows_u.at[sl], sem))
        copies.append(pltpu.async_copy(vb_hbm.at[idx_c.at[j]], vbv.at[sl], sem))
        copies.append(pltpu.async_copy(ub_hbm.at[idx_t.at[j]], ubv.at[sl], sem))
    copies.append(pltpu.async_copy(co_hbm.at[wid], cov, sem))
    copies.append(pltpu.async_copy(wt_hbm.at[wid], wtv, sem))
    for cp in copies:
        cp.wait()

    def body(g, lacc):
        rows16 = g * L + lax.iota(jnp.int32, L)
        acc = jnp.zeros((L,), jnp.float32)
        for c in range(EMB // 2):
            col = jnp.full((L,), c, jnp.int32)
            pv = plsc.load_gather(rows_v, [rows16, col])
            pu = plsc.load_gather(rows_u, [rows16, col])
            av, bv = plsc.unpack(plsc.bitcast(pv, jnp.bfloat16),
                                 format=plsc.PackFormat.INTERLEAVED)
            au, bu = plsc.unpack(plsc.bitcast(pu, jnp.bfloat16),
                                 format=plsc.PackFormat.INTERLEAVED)
            acc = acc + av * au + bv * bu
        sl = pl.ds(g * L, L)
        r = acc + vbv[sl] + ubv[sl] - cov[sl]
        return lacc + wtv[sl] * r * r

    accv[...] = lax.fori_loop(0, G, body, jnp.zeros((L,), jnp.float32))
    pltpu.sync_copy(accv, out_hbm.at[wid])


@jax.jit
def _sc_partials(c, t, co, wt, v_rows, u_rows, vb, ub):
    mesh = plsc.VectorSubcoreMesh(core_axis_name="c", subcore_axis_name="s")
    return pl.kernel(
        _sc_body,
        mesh=mesh,
        compiler_params=pltpu.CompilerParams(
            needs_layout_passes=False, use_tc_tiling_on_sc=False),
        out_type=jax.ShapeDtypeStruct((NW, L), jnp.float32),
        scratch_types=[
            pltpu.VMEM((NCH, CHUNK), jnp.int32),
            pltpu.VMEM((NCH, CHUNK), jnp.int32),
            pltpu.VMEM((BPW, EMB // 2), jnp.int32),
            pltpu.VMEM((BPW, EMB // 2), jnp.int32),
            pltpu.VMEM((BPW,), jnp.float32),
            pltpu.VMEM((BPW,), jnp.float32),
            pltpu.VMEM((BPW,), jnp.float32),
            pltpu.VMEM((BPW,), jnp.float32),
            pltpu.VMEM((L,), jnp.float32),
            pltpu.SemaphoreType.DMA,
        ],
    )(c, t, co, wt, v_rows, u_rows, vb, ub)


def _finish_body(x_ref, o_ref):
    o_ref[...] = jnp.sum(x_ref[...])[None, None]


def _finish(partials):
    return pl.pallas_call(
        _finish_body,
        out_shape=jax.ShapeDtypeStruct((1, 1), jnp.float32),
    )(partials)


def kernel(center_words, target_words, coocs, weighting, v_embed, u_embed,
           v_bias, u_bias):
    c = center_words.astype(jnp.int32).reshape(NW, NCH, CHUNK)
    t = target_words.astype(jnp.int32).reshape(NW, NCH, CHUNK)
    co = coocs.reshape(NW, BPW)
    wt = weighting.reshape(NW, BPW)
    vb = v_bias.reshape(VOCAB)
    ub = u_bias.reshape(VOCAB)
    v_rows = _repack(v_embed)
    u_rows = _repack(u_embed)
    partials = _sc_partials(c, t, co, wt, v_rows, u_rows, vb, ub)
    return _finish(partials)[0, 0]


# speedup vs baseline: 2.8414x; 2.8414x over previous
import jax
import jax.numpy as jnp
from jax import lax
from jax.experimental import pallas as pl
from jax.experimental.pallas import tpu as pltpu
from jax.experimental.pallas import tpu_sc as plsc

VOCAB = 1000000
EMB = 32
BATCH = 16384

NC = 2
NS = 16
L = 16
NW = NC * NS
BPW = BATCH // NW
CHUNK = 128
NCH = BPW // CHUNK
G = BPW // L


def _sc_body(c_hbm, t_hbm, co_hbm, wt_hbm, v_hbm, u_hbm, vb_hbm, ub_hbm,
             out_hbm, idx_c, idx_t, rows_v, rows_u, vbv, ubv, cov, wtv,
             accv, sem):
    wid = lax.axis_index("s") * NC + lax.axis_index("c")

    pltpu.sync_copy(c_hbm.at[wid], idx_c)
    pltpu.sync_copy(t_hbm.at[wid], idx_t)

    copies = []
    for j in range(NCH):
        sl = pl.ds(j * CHUNK, CHUNK)
        copies.append(pltpu.async_copy(v_hbm.at[idx_c.at[j]], rows_v.at[sl], sem))
        copies.append(pltpu.async_copy(u_hbm.at[idx_t.at[j]], rows_u.at[sl], sem))
        copies.append(pltpu.async_copy(vb_hbm.at[idx_c.at[j]], vbv.at[sl], sem))
        copies.append(pltpu.async_copy(ub_hbm.at[idx_t.at[j]], ubv.at[sl], sem))
    copies.append(pltpu.async_copy(co_hbm.at[wid], cov, sem))
    copies.append(pltpu.async_copy(wt_hbm.at[wid], wtv, sem))
    for cp in copies:
        cp.wait()

    def body(g, lacc):
        rows16 = g * L + lax.iota(jnp.int32, L)
        acc = jnp.zeros((L,), jnp.float32)
        for d in range(EMB):
            col = jnp.full((L,), d, jnp.int32)
            vd = plsc.load_gather(rows_v, [rows16, col])
            ud = plsc.load_gather(rows_u, [rows16, col])
            acc = acc + vd * ud
        sl = pl.ds(g * L, L)
        r = acc + vbv[sl] + ubv[sl] - cov[sl]
        return lacc + wtv[sl] * r * r

    accv[...] = lax.fori_loop(0, G, body, jnp.zeros((L,), jnp.float32))
    pltpu.sync_copy(accv, out_hbm.at[wid])


@jax.jit
def _sc_partials(c, t, co, wt, v_embed, u_embed, vb, ub):
    mesh = plsc.VectorSubcoreMesh(core_axis_name="c", subcore_axis_name="s")
    return pl.kernel(
        _sc_body,
        mesh=mesh,
        compiler_params=pltpu.CompilerParams(
            needs_layout_passes=False, use_tc_tiling_on_sc=False),
        out_type=jax.ShapeDtypeStruct((NW, L), jnp.float32),
        scratch_types=[
            pltpu.VMEM((NCH, CHUNK), jnp.int32),
            pltpu.VMEM((NCH, CHUNK), jnp.int32),
            pltpu.VMEM((BPW, EMB), jnp.float32),
            pltpu.VMEM((BPW, EMB), jnp.float32),
            pltpu.VMEM((BPW,), jnp.float32),
            pltpu.VMEM((BPW,), jnp.float32),
            pltpu.VMEM((BPW,), jnp.float32),
            pltpu.VMEM((BPW,), jnp.float32),
            pltpu.VMEM((L,), jnp.float32),
            pltpu.SemaphoreType.DMA,
        ],
    )(c, t, co, wt, v_embed, u_embed, vb, ub)


def _finish_body(x_ref, o_ref):
    o_ref[...] = jnp.sum(x_ref[...])[None, None]


def _finish(partials):
    return pl.pallas_call(
        _finish_body,
        out_shape=jax.ShapeDtypeStruct((1, 1), jnp.float32),
    )(partials)


def kernel(center_words, target_words, coocs, weighting, v_embed, u_embed,
           v_bias, u_bias):
    c = center_words.astype(jnp.int32).reshape(NW, NCH, CHUNK)
    t = target_words.astype(jnp.int32).reshape(NW, NCH, CHUNK)
    co = coocs.reshape(NW, BPW)
    wt = weighting.reshape(NW, BPW)
    vb = v_bias.reshape(VOCAB)
    ub = u_bias.reshape(VOCAB)
    partials = _sc_partials(c, t, co, wt, v_embed, u_embed, vb, ub)
    return _finish(partials)[0, 0]
